# baseline (speedup 1.0000x reference)
import functools

import jax
import jax.numpy as jnp
from jax import lax
from jax.experimental import pallas as pl
from jax.experimental.pallas import tpu as pltpu
from jax.experimental.pallas import tpu_sc as plsc

EMB = 32
BLK = 256
K = 5
VB = 16384
LB = 10
NCH = 4


@functools.cache
def _make_sc_gather(n_chunk: int, v_pad: int, b: int, l_chunk: int):
    info = plsc.get_sparse_core_info()
    nc, ns = info.num_cores, info.num_subcores
    nw = nc * ns
    per_w = n_chunk // nw
    n_blk = per_w // BLK
    m = b // 4
    assert per_w * nw == n_chunk and n_blk * BLK == per_w and n_blk % K == 0
    assert m % BLK == 0
    mesh = plsc.VectorSubcoreMesh(core_axis_name="c", subcore_axis_name="s")

    @functools.partial(
        pl.kernel,
        mesh=mesh,
        out_type=jax.ShapeDtypeStruct((l_chunk, m, 4 * EMB), jnp.float32),
        compiler_params=pltpu.CompilerParams(use_tc_tiling_on_sc=False),
        scratch_types=(
            [pltpu.VMEM((n_blk, BLK), jnp.int32),
             pltpu.VMEM((K, BLK, EMB), jnp.float32)]
            + [pltpu.SemaphoreType.DMA] * (2 * K)
        ),
    )
    def emb_gather(idx_hbm, table_hbm, out_hbm, idx_v, rows_v, *sems):
        gsem, ssem = sems[:K], sems[K:]
        wid = lax.axis_index("s") * nc + lax.axis_index("c")
        base = wid * per_w
        pltpu.sync_copy(idx_hbm.at[wid], idx_v)

        def store(bi, n0):
            li = n0 // b
            r = n0 % b
            return pltpu.async_copy(
                rows_v.at[bi],
                out_hbm.at[li, pl.ds(r % m, BLK), pl.ds((r // m) * EMB, EMB)],
                ssem[bi])

        def body(j, carry):
            g0 = j * K
            gh = [
                pltpu.async_copy(
                    table_hbm.at[idx_v.at[g0 + bi]], rows_v.at[bi], gsem[bi])
                for bi in range(K)
            ]
            sh = []
            for bi in range(K):
                gh[bi].wait()
                sh.append(store(bi, base + (g0 + bi) * BLK))
            for bi in range(K):
                sh[bi].wait()
            return carry

        lax.fori_loop(0, n_blk // K, body, 0)

    return emb_gather, nw, n_blk


def _wconv_body(in_ref, out_ref):
    x = in_ref[...]
    q = VB // 4
    zz = jnp.concatenate([x[:, q * j:q * (j + 1)] for j in range(4)], axis=0)
    out_ref[...] = zz.T


def _outconv_body(acc_ref, in_ref, out_ref):
    del acc_ref
    for li in range(LB):
        y = in_ref[li]
        z = y.T
        m = z.shape[1]
        for j in range(4):
            out_ref[li, :, m * j:m * (j + 1)] = z[EMB * j:EMB * (j + 1), :]


def kernel(seq, W):
    b, l = seq.shape
    vocab, emb = W.shape
    l_chunk = l // NCH
    assert emb == EMB and b % 4 == 0 and l_chunk * NCH == l and l_chunk % LB == 0
    m = b // 4
    n_chunk = l_chunk * b

    n_wblk = -(-vocab // VB)
    v_pad = n_wblk * VB

    idx = seq.astype(jnp.int32)
    w_loc = idx % VB
    idx = (idx - w_loc) + 4 * (w_loc % (VB // 4)) + w_loc // (VB // 4)
    idx = idx.T.reshape(-1)

    w_rm2d = pl.pallas_call(
        _wconv_body,
        grid=(n_wblk,),
        in_specs=[pl.BlockSpec((EMB, VB), lambda i: (0, i))],
        out_specs=pl.BlockSpec((VB // 4, 128), lambda i: (i, 0)),
        out_shape=jax.ShapeDtypeStruct((v_pad // 4, 128), jnp.float32),
    )(W.T)
    w_rm = w_rm2d.reshape(v_pad, EMB)

    emb_gather, nw, n_blk = _make_sc_gather(n_chunk, v_pad, b, l_chunk)
    idx_ch = idx.reshape(NCH, nw, n_blk, BLK)
    gathered = [emb_gather(idx_ch[k], w_rm) for k in range(NCH)]

    out = None
    nblk_l = l_chunk // LB
    for k in range(NCH):
        if k == 0:
            body = functools.partial(_outconv_body, None)
            in_specs = []
            aliases = {}
            args = ()
        else:
            body = _outconv_body
            in_specs = [pl.BlockSpec(memory_space=pl.ANY)]
            aliases = {0: 0}
            args = (out,)
        out = pl.pallas_call(
            body,
            grid=(nblk_l,),
            in_specs=in_specs + [pl.BlockSpec((LB, m, 128), lambda i: (i, 0, 0))],
            out_specs=pl.BlockSpec(
                (LB, EMB, b),
                functools.partial(lambda k_, i: (k_ * nblk_l + i, 0, 0), k)),
            out_shape=jax.ShapeDtypeStruct((l, EMB, b), jnp.float32),
            input_output_aliases=aliases,
        )(*args, gathered[k].reshape(l_chunk, m, 128))

    return out.transpose(2, 0, 1)


# speedup vs baseline: 1.0431x; 1.0431x over previous
import functools

import jax
import jax.numpy as jnp
from jax import lax
from jax.experimental import pallas as pl
from jax.experimental.pallas import tpu as pltpu
from jax.experimental.pallas import tpu_sc as plsc

EMB = 32
BLK = 512
K = 5
VB = 16384
LB = 10
NCH = 1


@functools.cache
def _make_sc_gather(n_chunk: int, v_pad: int, b: int, l_chunk: int):
    info = plsc.get_sparse_core_info()
    nc, ns = info.num_cores, info.num_subcores
    nw = nc * ns
    per_w = n_chunk // nw
    n_blk = per_w // BLK
    m = b // 4
    assert per_w * nw == n_chunk and n_blk * BLK == per_w and n_blk % K == 0
    assert m % BLK == 0
    mesh = plsc.VectorSubcoreMesh(core_axis_name="c", subcore_axis_name="s")

    @functools.partial(
        pl.kernel,
        mesh=mesh,
        out_type=jax.ShapeDtypeStruct((l_chunk, m, 4 * EMB), jnp.float32),
        compiler_params=pltpu.CompilerParams(use_tc_tiling_on_sc=False),
        scratch_types=(
            [pltpu.VMEM((n_blk, BLK), jnp.int32),
             pltpu.VMEM((K, BLK, EMB), jnp.float32)]
            + [pltpu.SemaphoreType.DMA] * (2 * K)
        ),
    )
    def emb_gather(idx_hbm, table_hbm, out_hbm, idx_v, rows_v, *sems):
        gsem, ssem = sems[:K], sems[K:]
        wid = lax.axis_index("s") * nc + lax.axis_index("c")
        base = wid * per_w
        pltpu.sync_copy(idx_hbm.at[wid], idx_v)

        def store(bi, n0):
            li = n0 // b
            r = n0 % b
            return pltpu.async_copy(
                rows_v.at[bi],
                out_hbm.at[li, pl.ds(r % m, BLK), pl.ds((r // m) * EMB, EMB)],
                ssem[bi])

        def body(j, carry):
            g0 = j * K
            gh = [
                pltpu.async_copy(
                    table_hbm.at[idx_v.at[g0 + bi]], rows_v.at[bi], gsem[bi])
                for bi in range(K)
            ]
            sh = []
            for bi in range(K):
                gh[bi].wait()
                sh.append(store(bi, base + (g0 + bi) * BLK))
            for bi in range(K):
                sh[bi].wait()
            return carry

        lax.fori_loop(0, n_blk // K, body, 0)

    return emb_gather, nw, n_blk


def _wconv_body(in_ref, out_ref):
    x = in_ref[...]
    q = VB // 4
    zz = jnp.concatenate([x[:, q * j:q * (j + 1)] for j in range(4)], axis=0)
    out_ref[...] = zz.T


def _outconv_body(acc_ref, in_ref, out_ref):
    del acc_ref
    for li in range(LB):
        y = in_ref[li]
        z = y.T
        m = z.shape[1]
        for j in range(4):
            out_ref[li, :, m * j:m * (j + 1)] = z[EMB * j:EMB * (j + 1), :]


def kernel(seq, W):
    b, l = seq.shape
    vocab, emb = W.shape
    l_chunk = l // NCH
    assert emb == EMB and b % 4 == 0 and l_chunk * NCH == l and l_chunk % LB == 0
    m = b // 4
    n_chunk = l_chunk * b

    n_wblk = -(-vocab // VB)
    v_pad = n_wblk * VB

    idx = seq.astype(jnp.int32)
    w_loc = idx % VB
    idx = (idx - w_loc) + 4 * (w_loc % (VB // 4)) + w_loc // (VB // 4)
    idx = idx.T.reshape(-1)

    w_rm2d = pl.pallas_call(
        _wconv_body,
        grid=(n_wblk,),
        in_specs=[pl.BlockSpec((EMB, VB), lambda i: (0, i))],
        out_specs=pl.BlockSpec((VB // 4, 128), lambda i: (i, 0)),
        out_shape=jax.ShapeDtypeStruct((v_pad // 4, 128), jnp.float32),
    )(W.T)
    w_rm = w_rm2d.reshape(v_pad, EMB)

    emb_gather, nw, n_blk = _make_sc_gather(n_chunk, v_pad, b, l_chunk)
    idx_ch = idx.reshape(NCH, nw, n_blk, BLK)
    gathered = [emb_gather(idx_ch[k], w_rm) for k in range(NCH)]

    out = None
    nblk_l = l_chunk // LB
    for k in range(NCH):
        if k == 0:
            body = functools.partial(_outconv_body, None)
            in_specs = []
            aliases = {}
            args = ()
        else:
            body = _outconv_body
            in_specs = [pl.BlockSpec(memory_space=pl.ANY)]
            aliases = {0: 0}
            args = (out,)
        out = pl.pallas_call(
            body,
            grid=(nblk_l,),
            in_specs=in_specs + [pl.BlockSpec((LB, m, 128), lambda i: (i, 0, 0))],
            out_specs=pl.BlockSpec(
                (LB, EMB, b),
                functools.partial(lambda k_, i: (k_ * nblk_l + i, 0, 0), k)),
            out_shape=jax.ShapeDtypeStruct((l, EMB, b), jnp.float32),
            input_output_aliases=aliases,
        )(*args, gathered[k].reshape(l_chunk, m, 128))

    return out.transpose(2, 0, 1)


# speedup vs baseline: 1.0488x; 1.0055x over previous
import functools

import jax
import jax.numpy as jnp
from jax import lax
from jax.experimental import pallas as pl
from jax.experimental.pallas import tpu as pltpu
from jax.experimental.pallas import tpu_sc as plsc

EMB = 32
BLK = 512
K = 5
VB = 16384
LB = 20
NCH = 1


@functools.cache
def _make_sc_gather(n_chunk: int, v_pad: int, b: int, l_chunk: int):
    info = plsc.get_sparse_core_info()
    nc, ns = info.num_cores, info.num_subcores
    nw = nc * ns
    per_w = n_chunk // nw
    n_blk = per_w // BLK
    m = b // 4
    assert per_w * nw == n_chunk and n_blk * BLK == per_w and n_blk % K == 0
    assert m % BLK == 0
    mesh = plsc.VectorSubcoreMesh(core_axis_name="c", subcore_axis_name="s")

    @functools.partial(
        pl.kernel,
        mesh=mesh,
        out_type=jax.ShapeDtypeStruct((l_chunk, m, 4 * EMB), jnp.float32),
        compiler_params=pltpu.CompilerParams(use_tc_tiling_on_sc=False),
        scratch_types=(
            [pltpu.VMEM((n_blk, BLK), jnp.int32),
             pltpu.VMEM((K, BLK, EMB), jnp.float32)]
            + [pltpu.SemaphoreType.DMA] * (2 * K)
        ),
    )
    def emb_gather(idx_hbm, table_hbm, out_hbm, idx_v, rows_v, *sems):
        gsem, ssem = sems[:K], sems[K:]
        wid = lax.axis_index("s") * nc + lax.axis_index("c")
        base = wid * per_w
        pltpu.sync_copy(idx_hbm.at[wid], idx_v)

        def store(bi, n0):
            li = n0 // b
            r = n0 % b
            return pltpu.async_copy(
                rows_v.at[bi],
                out_hbm.at[li, pl.ds(r % m, BLK), pl.ds((r // m) * EMB, EMB)],
                ssem[bi])

        def body(j, carry):
            g0 = j * K
            gh = [
                pltpu.async_copy(
                    table_hbm.at[idx_v.at[g0 + bi]], rows_v.at[bi], gsem[bi])
                for bi in range(K)
            ]
            sh = []
            for bi in range(K):
                gh[bi].wait()
                sh.append(store(bi, base + (g0 + bi) * BLK))
            for bi in range(K):
                sh[bi].wait()
            return carry

        lax.fori_loop(0, n_blk // K, body, 0)

    return emb_gather, nw, n_blk


def _wconv_body(in_ref, out_ref):
    x = in_ref[...]
    q = VB // 4
    zz = jnp.concatenate([x[:, q * j:q * (j + 1)] for j in range(4)], axis=0)
    out_ref[...] = zz.T


def _outconv_body(acc_ref, in_ref, out_ref):
    del acc_ref
    for li in range(LB):
        y = in_ref[li]
        z = y.T
        m = z.shape[1]
        for j in range(4):
            out_ref[li, :, m * j:m * (j + 1)] = z[EMB * j:EMB * (j + 1), :]


def kernel(seq, W):
    b, l = seq.shape
    vocab, emb = W.shape
    l_chunk = l // NCH
    assert emb == EMB and b % 4 == 0 and l_chunk * NCH == l and l_chunk % LB == 0
    m = b // 4
    n_chunk = l_chunk * b

    n_wblk = -(-vocab // VB)
    v_pad = n_wblk * VB

    idx = seq.astype(jnp.int32)
    w_loc = idx % VB
    idx = (idx - w_loc) + 4 * (w_loc % (VB // 4)) + w_loc // (VB // 4)
    idx = idx.T.reshape(-1)

    w_rm2d = pl.pallas_call(
        _wconv_body,
        grid=(n_wblk,),
        in_specs=[pl.BlockSpec((EMB, VB), lambda i: (0, i))],
        out_specs=pl.BlockSpec((VB // 4, 128), lambda i: (i, 0)),
        out_shape=jax.ShapeDtypeStruct((v_pad // 4, 128), jnp.float32),
    )(W.T)
    w_rm = w_rm2d.reshape(v_pad, EMB)

    emb_gather, nw, n_blk = _make_sc_gather(n_chunk, v_pad, b, l_chunk)
    idx_ch = idx.reshape(NCH, nw, n_blk, BLK)
    gathered = [emb_gather(idx_ch[k], w_rm) for k in range(NCH)]

    out = None
    nblk_l = l_chunk // LB
    for k in range(NCH):
        if k == 0:
            body = functools.partial(_outconv_body, None)
            in_specs = []
            aliases = {}
            args = ()
        else:
            body = _outconv_body
            in_specs = [pl.BlockSpec(memory_space=pl.ANY)]
            aliases = {0: 0}
            args = (out,)
        out = pl.pallas_call(
            body,
            grid=(nblk_l,),
            in_specs=in_specs + [pl.BlockSpec((LB, m, 128), lambda i: (i, 0, 0))],
            out_specs=pl.BlockSpec(
                (LB, EMB, b),
                functools.partial(lambda k_, i: (k_ * nblk_l + i, 0, 0), k)),
            out_shape=jax.ShapeDtypeStruct((l, EMB, b), jnp.float32),
            input_output_aliases=aliases,
        )(*args, gathered[k].reshape(l_chunk, m, 128))

    return out.transpose(2, 0, 1)
